# Initial kernel scaffold; baseline (speedup 1.0000x reference)
import jax
import jax.numpy as jnp
from jax.experimental import pallas as pl


def kernel(h, node_embeddings, edge_index, Wq, bq, Wk, bk, u_pos):
    raise NotImplementedError("write your pallas kernel here")


# speedup vs baseline: 1.1709x; 1.1709x over previous
import functools
import math

import jax
import jax.numpy as jnp
from jax import lax
from jax.experimental import pallas as pl
from jax.experimental.pallas import tpu as pltpu
from jax.experimental.pallas import tpu_sc as plsc

N = 10000
E = 320000
D = 128
W = 2 * D
BETA_POS = 0.1

NC = 2
NS = 16
NW = NC * NS
ESUB = E // NW
C = 80
NCHUNK = ESUB // C
G = C // 16

_R = 1000


def _prep_body(h_ref, emb_ref, wq_ref, bq_ref, wk_ref, bk_ref, up_ref,
               u_ref, v_ref, b_ref):
    hh = h_ref[...]
    ee = emb_ref[...]
    dn = (((1,), (1,)), ((), ()))
    q = lax.dot_general(hh, wq_ref[...], dn,
                        preferred_element_type=jnp.float32,
                        precision=lax.Precision.HIGHEST) + bq_ref[...]
    k = lax.dot_general(hh, wk_ref[...], dn,
                        preferred_element_type=jnp.float32,
                        precision=lax.Precision.HIGHEST) + bk_ref[...]
    u_ref[:, :D] = ee
    u_ref[:, D:] = q
    v_ref[:, :D] = ee
    v_ref[:, D:] = k
    b_ref[...] = lax.dot_general(ee, up_ref[...], (((1,), (0,)), ((), ())),
                                 preferred_element_type=jnp.float32,
                                 precision=lax.Precision.HIGHEST)


def _prep(h, emb, Wq, bq2, Wk, bk2, up2):
    grid = (N // _R,)
    row = lambda i: (i, 0)
    full = lambda i: (0, 0)
    return pl.pallas_call(
        _prep_body,
        grid=grid,
        in_specs=[
            pl.BlockSpec((_R, D), row),
            pl.BlockSpec((_R, D), row),
            pl.BlockSpec((D, D), full),
            pl.BlockSpec((1, D), full),
            pl.BlockSpec((D, D), full),
            pl.BlockSpec((1, D), full),
            pl.BlockSpec((D, 1), full),
        ],
        out_specs=[
            pl.BlockSpec((_R, W), row),
            pl.BlockSpec((_R, W), row),
            pl.BlockSpec((_R, 1), row),
        ],
        out_shape=[
            jax.ShapeDtypeStruct((N, W), jnp.float32),
            jax.ShapeDtypeStruct((N, W), jnp.float32),
            jax.ShapeDtypeStruct((N, 1), jnp.float32),
        ],
    )(h, emb, Wq, bq2, Wk, bk2, up2)


def _edge_body(u_hbm, v_hbm, b_hbm, src_hbm, dst_hbm,
               out_e, out_a, out_q,
               b_v, src_v, dst_v, u_rows, v_rows, oe_v, oa_v, oq_v, sem):
    wid = lax.axis_index("s") * NC + lax.axis_index("c")
    base = wid * ESUB
    pltpu.sync_copy(b_hbm, b_v)

    def chunk_body(ci, carry):
        off = base + ci * C
        pltpu.sync_copy(src_hbm.at[pl.ds(off, C)], src_v)
        pltpu.sync_copy(dst_hbm.at[pl.ds(off, C)], dst_v)
        cp1 = pltpu.async_copy(u_hbm.at[src_v], u_rows, sem)
        cp2 = pltpu.async_copy(v_hbm.at[dst_v], v_rows, sem)
        cp1.wait()
        cp2.wait()

        def group_body(g, gcarry):
            e0 = g * 16
            rows16 = e0 + lax.broadcasted_iota(jnp.int32, (16,), 0)
            s16 = src_v[pl.ds(e0, 16)]
            d16 = dst_v[pl.ds(e0, 16)]
            bs = plsc.load_gather(b_v, [s16])
            bd = plsc.load_gather(b_v, [d16])

            def h1(dd, acc):
                col = jnp.full((16,), dd, jnp.int32)
                uu = plsc.load_gather(u_rows, [rows16, col])
                vv = plsc.load_gather(v_rows, [rows16, col])
                diff = vv - uu
                return acc + diff * diff

            euc2 = lax.fori_loop(0, D, h1, jnp.zeros((16,), jnp.float32))

            def h2(dd, acc):
                col = jnp.full((16,), dd, jnp.int32)
                uu = plsc.load_gather(u_rows, [rows16, col])
                vv = plsc.load_gather(v_rows, [rows16, col])
                return acc + uu * vv

            qk = lax.fori_loop(D, W, h2, jnp.zeros((16,), jnp.float32))

            oe_v[pl.ds(e0, 16)] = euc2
            oa_v[pl.ds(e0, 16)] = bd - bs
            oq_v[pl.ds(e0, 16)] = qk
            return gcarry

        lax.fori_loop(0, G, group_body, 0)
        pltpu.sync_copy(oe_v, out_e.at[pl.ds(off, C)])
        pltpu.sync_copy(oa_v, out_a.at[pl.ds(off, C)])
        pltpu.sync_copy(oq_v, out_q.at[pl.ds(off, C)])
        return carry

    lax.fori_loop(0, NCHUNK, chunk_body, 0)


_edges = pl.kernel(
    _edge_body,
    out_type=[
        jax.ShapeDtypeStruct((E,), jnp.float32),
        jax.ShapeDtypeStruct((E,), jnp.float32),
        jax.ShapeDtypeStruct((E,), jnp.float32),
    ],
    mesh=plsc.VectorSubcoreMesh(core_axis_name="c", subcore_axis_name="s"),
    compiler_params=pltpu.CompilerParams(needs_layout_passes=False),
    scratch_types=[
        pltpu.VMEM((N,), jnp.float32),
        pltpu.VMEM((C,), jnp.int32),
        pltpu.VMEM((C,), jnp.int32),
        pltpu.VMEM((C, W), jnp.float32),
        pltpu.VMEM((C, W), jnp.float32),
        pltpu.VMEM((C,), jnp.float32),
        pltpu.VMEM((C,), jnp.float32),
        pltpu.VMEM((C,), jnp.float32),
        pltpu.SemaphoreType.DMA,
    ],
)


def _softmax_body(e_ref, a_ref, q_ref, o_ref):
    d_fk = jnp.sqrt(e_ref[...]) + BETA_POS * a_ref[...]
    s = jnp.exp(-d_fk) * q_ref[...] * (1.0 / math.sqrt(float(D)))
    m = jnp.max(s)
    ex = jnp.exp(s - m)
    o_ref[...] = ex / jnp.sum(ex)


_SR = 2500
_SC_COLS = 128


def _softmax(e2, a2, q2):
    return pl.pallas_call(
        _softmax_body,
        out_shape=jax.ShapeDtypeStruct((_SR, _SC_COLS), jnp.float32),
    )(e2, a2, q2)


def kernel(h, node_embeddings, edge_index, Wq, bq, Wk, bk, u_pos):
    src = edge_index[0]
    dst = edge_index[1]
    U, V, b2 = _prep(h, node_embeddings, Wq, bq.reshape(1, D),
                     Wk, bk.reshape(1, D), u_pos.reshape(D, 1))
    b = b2.reshape(N)
    euc2, asym, qk = _edges(U, V, b, src, dst)
    att = _softmax(euc2.reshape(_SR, _SC_COLS),
                   asym.reshape(_SR, _SC_COLS),
                   qk.reshape(_SR, _SC_COLS))
    return att.reshape(E)


# speedup vs baseline: 4.1617x; 3.5543x over previous
import functools
import math

import jax
import jax.numpy as jnp
from jax import lax
from jax.experimental import pallas as pl
from jax.experimental.pallas import tpu as pltpu
from jax.experimental.pallas import tpu_sc as plsc

N = 10000
E = 320000
D = 128
W = 2 * D
BETA_POS = 0.1

NC = 2
NS = 16
NW = NC * NS
ESUB = E // NW
C = 32
NFULL = ESUB // C
PAIRS = NFULL // 2
TAIL = ESUB - NFULL * C

_R = 1000


def _prep_body(h_ref, emb_ref, wq_ref, bq_ref, wk_ref, bk_ref, up_ref,
               u_ref, v_ref, b_ref):
    hh = h_ref[...]
    ee = emb_ref[...]
    dn = (((1,), (1,)), ((), ()))
    q = lax.dot_general(hh, wq_ref[...], dn,
                        preferred_element_type=jnp.float32,
                        precision=lax.Precision.HIGHEST) + bq_ref[...]
    k = lax.dot_general(hh, wk_ref[...], dn,
                        preferred_element_type=jnp.float32,
                        precision=lax.Precision.HIGHEST) + bk_ref[...]
    u_ref[:, :D] = ee
    u_ref[:, D:] = q
    v_ref[:, :D] = ee
    v_ref[:, D:] = k
    b_ref[...] = lax.dot_general(ee, up_ref[...], (((1,), (0,)), ((), ())),
                                 preferred_element_type=jnp.float32,
                                 precision=lax.Precision.HIGHEST)


def _prep(h, emb, Wq, bq2, Wk, bk2, up2):
    grid = (N // _R,)
    row = lambda i: (i, 0)
    full = lambda i: (0, 0)
    return pl.pallas_call(
        _prep_body,
        grid=grid,
        in_specs=[
            pl.BlockSpec((_R, D), row),
            pl.BlockSpec((_R, D), row),
            pl.BlockSpec((D, D), full),
            pl.BlockSpec((1, D), full),
            pl.BlockSpec((D, D), full),
            pl.BlockSpec((1, D), full),
            pl.BlockSpec((D, 1), full),
        ],
        out_specs=[
            pl.BlockSpec((_R, W), row),
            pl.BlockSpec((_R, W), row),
            pl.BlockSpec((_R, 1), row),
        ],
        out_shape=[
            jax.ShapeDtypeStruct((N, W), jnp.float32),
            jax.ShapeDtypeStruct((N, W), jnp.float32),
            jax.ShapeDtypeStruct((N, 1), jnp.float32),
        ],
    )(h, emb, Wq, bq2, Wk, bk2, up2)


def _edge_body(u_hbm, v_hbm, b_hbm, src_hbm, dst_hbm,
               out_e, out_a, out_q,
               b_v, src_all, dst_all, u0, v0, u1, v1,
               oe_v, oa_v, oq_v, sem0, sem1):
    wid = lax.axis_index("s") * NC + lax.axis_index("c")
    base = wid * ESUB
    pltpu.sync_copy(b_hbm, b_v)
    pltpu.sync_copy(src_hbm.at[pl.ds(base, ESUB)], src_all)
    pltpu.sync_copy(dst_hbm.at[pl.ds(base, ESUB)], dst_all)

    def fire(ci, ub, vb, sem):
        idx_s = src_all.at[pl.ds(ci * C, C)]
        idx_d = dst_all.at[pl.ds(ci * C, C)]
        pltpu.async_copy(u_hbm.at[idx_s], ub, sem)
        pltpu.async_copy(v_hbm.at[idx_d], vb, sem)

    def wait(ub, vb, sem):
        pltpu.make_async_copy(u_hbm.at[src_all.at[pl.ds(0, C)]], ub, sem).wait()
        pltpu.make_async_copy(v_hbm.at[dst_all.at[pl.ds(0, C)]], vb, sem).wait()

    lane = lax.broadcasted_iota(jnp.int32, (16,), 0)

    def group16(base_e, ub, vb, row0):
        res_e = jnp.zeros((16,), jnp.float32)
        res_q = jnp.zeros((16,), jnp.float32)
        for j in range(16):
            r = row0 + j
            acc_e = acc_q = None
            for k in range(8):
                uu = ub[r, pl.ds(k * 16, 16)]
                vv = vb[r, pl.ds(k * 16, 16)]
                dd = vv - uu
                acc_e = dd * dd if acc_e is None else acc_e + dd * dd
            for k in range(8, 16):
                uu = ub[r, pl.ds(k * 16, 16)]
                vv = vb[r, pl.ds(k * 16, 16)]
                acc_q = uu * vv if acc_q is None else acc_q + uu * vv
            mask = lane == j
            res_e = jnp.where(mask, jnp.sum(acc_e), res_e)
            res_q = jnp.where(mask, jnp.sum(acc_q), res_q)
        oe_v[pl.ds(base_e, 16)] = res_e
        oq_v[pl.ds(base_e, 16)] = res_q

    def compute(ci, ub, vb):
        o0 = ci * C

        def gbody(g, carry):
            group16(o0 + g * 16, ub, vb, g * 16)
            return carry

        lax.fori_loop(0, C // 16, gbody, 0)

    def asym_body(g, carry):
        e0 = g * 16
        s16 = src_all[pl.ds(e0, 16)]
        d16 = dst_all[pl.ds(e0, 16)]
        bs = plsc.load_gather(b_v, [s16])
        bd = plsc.load_gather(b_v, [d16])
        oa_v[pl.ds(e0, 16)] = bd - bs
        return carry

    lax.fori_loop(0, ESUB // 16, asym_body, 0)

    fire(0, u0, v0, sem0)

    def pair_body(p, carry):
        ci0 = 2 * p
        fire(ci0 + 1, u1, v1, sem1)
        wait(u0, v0, sem0)
        compute(ci0, u0, v0)

        @pl.when(p < PAIRS - 1)
        def _():
            fire(ci0 + 2, u0, v0, sem0)

        wait(u1, v1, sem1)
        compute(ci0 + 1, u1, v1)
        return carry

    lax.fori_loop(0, PAIRS, pair_body, 0)

    t0 = NFULL * C
    pltpu.async_copy(u_hbm.at[src_all.at[pl.ds(t0, TAIL)]],
                     u0.at[pl.ds(0, TAIL)], sem0)
    pltpu.async_copy(v_hbm.at[dst_all.at[pl.ds(t0, TAIL)]],
                     v0.at[pl.ds(0, TAIL)], sem0)
    pltpu.make_async_copy(u_hbm.at[src_all.at[pl.ds(t0, TAIL)]],
                          u0.at[pl.ds(0, TAIL)], sem0).wait()
    pltpu.make_async_copy(v_hbm.at[dst_all.at[pl.ds(t0, TAIL)]],
                          v0.at[pl.ds(0, TAIL)], sem0).wait()
    group16(t0, u0, v0, 0)

    pltpu.sync_copy(oe_v, out_e.at[pl.ds(base, ESUB)])
    pltpu.sync_copy(oa_v, out_a.at[pl.ds(base, ESUB)])
    pltpu.sync_copy(oq_v, out_q.at[pl.ds(base, ESUB)])


_edges = pl.kernel(
    _edge_body,
    out_type=[
        jax.ShapeDtypeStruct((E,), jnp.float32),
        jax.ShapeDtypeStruct((E,), jnp.float32),
        jax.ShapeDtypeStruct((E,), jnp.float32),
    ],
    mesh=plsc.VectorSubcoreMesh(core_axis_name="c", subcore_axis_name="s"),
    compiler_params=pltpu.CompilerParams(needs_layout_passes=False),
    scratch_types=[
        pltpu.VMEM((N,), jnp.float32),
        pltpu.VMEM((ESUB,), jnp.int32),
        pltpu.VMEM((ESUB,), jnp.int32),
        pltpu.VMEM((C, W), jnp.float32),
        pltpu.VMEM((C, W), jnp.float32),
        pltpu.VMEM((C, W), jnp.float32),
        pltpu.VMEM((C, W), jnp.float32),
        pltpu.VMEM((ESUB,), jnp.float32),
        pltpu.VMEM((ESUB,), jnp.float32),
        pltpu.VMEM((ESUB,), jnp.float32),
        pltpu.SemaphoreType.DMA,
        pltpu.SemaphoreType.DMA,
    ],
)


def _softmax_body(e_ref, a_ref, q_ref, o_ref):
    d_fk = jnp.sqrt(e_ref[...]) + BETA_POS * a_ref[...]
    s = jnp.exp(-d_fk) * q_ref[...] * (1.0 / math.sqrt(float(D)))
    m = jnp.max(s)
    ex = jnp.exp(s - m)
    o_ref[...] = ex / jnp.sum(ex)


_SR = 2500
_SC_COLS = 128


def _softmax(e2, a2, q2):
    return pl.pallas_call(
        _softmax_body,
        out_shape=jax.ShapeDtypeStruct((_SR, _SC_COLS), jnp.float32),
    )(e2, a2, q2)


def kernel(h, node_embeddings, edge_index, Wq, bq, Wk, bk, u_pos):
    src = edge_index[0]
    dst = edge_index[1]
    U, V, b2 = _prep(h, node_embeddings, Wq, bq.reshape(1, D),
                     Wk, bk.reshape(1, D), u_pos.reshape(D, 1))
    b = b2.reshape(N)
    euc2, asym, qk = _edges(U, V, b, src, dst)
    att = _softmax(euc2.reshape(_SR, _SC_COLS),
                   asym.reshape(_SR, _SC_COLS),
                   qk.reshape(_SR, _SC_COLS))
    return att.reshape(E)


# speedup vs baseline: 13.3798x; 3.2150x over previous
import functools
import math

import jax
import jax.numpy as jnp
from jax import lax
from jax.experimental import pallas as pl
from jax.experimental.pallas import tpu as pltpu
from jax.experimental.pallas import tpu_sc as plsc

N = 10000
E = 320000
D = 128
W = 2 * D
WP = D
BETA_POS = 0.1

NC = 2
NS = 16
NW = NC * NS
ESUB = E // NW
C = 64
NFULL = ESUB // C
PAIRS = NFULL // 2
TAIL = ESUB - NFULL * C

_R = 1000


def _prep_body(h_ref, emb_ref, wq_ref, bq_ref, wk_ref, bk_ref, up_ref,
               u_ref, v_ref, b_ref):
    hh = h_ref[...]
    ee = emb_ref[...]
    dn = (((1,), (1,)), ((), ()))
    q = lax.dot_general(hh, wq_ref[...], dn,
                        preferred_element_type=jnp.float32,
                        precision=lax.Precision.HIGHEST) + bq_ref[...]
    k = lax.dot_general(hh, wk_ref[...], dn,
                        preferred_element_type=jnp.float32,
                        precision=lax.Precision.HIGHEST) + bk_ref[...]
    def pack_pair(xlo, xhi):
        lo = lax.bitcast_convert_type(xlo.astype(jnp.bfloat16),
                                      jnp.uint16).astype(jnp.uint32)
        hi = lax.bitcast_convert_type(xhi.astype(jnp.bfloat16),
                                      jnp.uint16).astype(jnp.uint32)
        return lax.bitcast_convert_type(lo | (hi << 16), jnp.int32)

    hd = D // 2
    u_ref[:, :hd] = pack_pair(ee[:, :hd], ee[:, hd:])
    u_ref[:, hd:] = pack_pair(q[:, :hd], q[:, hd:])
    v_ref[:, :hd] = pack_pair(ee[:, :hd], ee[:, hd:])
    v_ref[:, hd:] = pack_pair(k[:, :hd], k[:, hd:])
    b_ref[...] = lax.dot_general(ee, up_ref[...], (((1,), (0,)), ((), ())),
                                 preferred_element_type=jnp.float32,
                                 precision=lax.Precision.HIGHEST)


def _prep(h, emb, Wq, bq2, Wk, bk2, up2):
    grid = (N // _R,)
    row = lambda i: (i, 0)
    full = lambda i: (0, 0)
    return pl.pallas_call(
        _prep_body,
        grid=grid,
        in_specs=[
            pl.BlockSpec((_R, D), row),
            pl.BlockSpec((_R, D), row),
            pl.BlockSpec((D, D), full),
            pl.BlockSpec((1, D), full),
            pl.BlockSpec((D, D), full),
            pl.BlockSpec((1, D), full),
            pl.BlockSpec((D, 1), full),
        ],
        out_specs=[
            pl.BlockSpec((_R, WP), row),
            pl.BlockSpec((_R, WP), row),
            pl.BlockSpec((_R, 1), row),
        ],
        out_shape=[
            jax.ShapeDtypeStruct((N, WP), jnp.int32),
            jax.ShapeDtypeStruct((N, WP), jnp.int32),
            jax.ShapeDtypeStruct((N, 1), jnp.float32),
        ],
    )(h, emb, Wq, bq2, Wk, bk2, up2)


def _edge_body(u_hbm, v_hbm, b_hbm, src_hbm, dst_hbm,
               out_e, out_a, out_q,
               b_v, src_all, dst_all, u0, v0, u1, v1,
               oe_v, oa_v, oq_v, sem0, sem1):
    wid = lax.axis_index("s") * NC + lax.axis_index("c")
    base = wid * ESUB
    pltpu.sync_copy(b_hbm, b_v)
    pltpu.sync_copy(src_hbm.at[pl.ds(base, ESUB)], src_all)
    pltpu.sync_copy(dst_hbm.at[pl.ds(base, ESUB)], dst_all)

    def fire(ci, ub, vb, sem):
        idx_s = src_all.at[pl.ds(ci * C, C)]
        idx_d = dst_all.at[pl.ds(ci * C, C)]
        pltpu.async_copy(u_hbm.at[idx_s], ub, sem)
        pltpu.async_copy(v_hbm.at[idx_d], vb, sem)

    def wait(ub, vb, sem):
        pltpu.make_async_copy(u_hbm.at[src_all.at[pl.ds(0, C)]], ub, sem).wait()
        pltpu.make_async_copy(v_hbm.at[dst_all.at[pl.ds(0, C)]], vb, sem).wait()

    lane = lax.broadcasted_iota(jnp.int32, (16,), 0)

    def expand(ref, r, k):
        w = ref[r, pl.ds(k * 16, 16)]
        hi = plsc.bitcast(w, jnp.float32)
        lo = plsc.bitcast(w << 16, jnp.float32)
        return hi, lo

    def group16(base_e, ub, vb, row0):
        res_e = jnp.zeros((16,), jnp.float32)
        res_q = jnp.zeros((16,), jnp.float32)
        for j in range(16):
            r = row0 + j
            acc_e = acc_q = None
            for k in range(4):
                uh, ul = expand(ub, r, k)
                vh, vl = expand(vb, r, k)
                dh = vh - uh
                dl = vl - ul
                t = dh * dh + dl * dl
                acc_e = t if acc_e is None else acc_e + t
            for k in range(4, 8):
                uh, ul = expand(ub, r, k)
                vh, vl = expand(vb, r, k)
                t = uh * vh + ul * vl
                acc_q = t if acc_q is None else acc_q + t
            mask = lane == j
            res_e = jnp.where(mask, jnp.sum(acc_e), res_e)
            res_q = jnp.where(mask, jnp.sum(acc_q), res_q)
        oe_v[pl.ds(base_e, 16)] = res_e
        oq_v[pl.ds(base_e, 16)] = res_q

    def compute(ci, ub, vb):
        o0 = ci * C

        def gbody(g, carry):
            group16(o0 + g * 16, ub, vb, g * 16)
            return carry

        lax.fori_loop(0, C // 16, gbody, 0)

    def asym_body(g, carry):
        e0 = g * 16
        s16 = src_all[pl.ds(e0, 16)]
        d16 = dst_all[pl.ds(e0, 16)]
        bs = plsc.load_gather(b_v, [s16])
        bd = plsc.load_gather(b_v, [d16])
        oa_v[pl.ds(e0, 16)] = bd - bs
        return carry

    lax.fori_loop(0, ESUB // 16, asym_body, 0)

    fire(0, u0, v0, sem0)

    def pair_body(p, carry):
        ci0 = 2 * p
        fire(ci0 + 1, u1, v1, sem1)
        wait(u0, v0, sem0)
        compute(ci0, u0, v0)

        @pl.when(p < PAIRS - 1)
        def _():
            fire(ci0 + 2, u0, v0, sem0)

        wait(u1, v1, sem1)
        compute(ci0 + 1, u1, v1)
        return carry

    lax.fori_loop(0, PAIRS, pair_body, 0)

    t0 = NFULL * C
    pltpu.async_copy(u_hbm.at[src_all.at[pl.ds(t0, TAIL)]],
                     u0.at[pl.ds(0, TAIL)], sem0)
    pltpu.async_copy(v_hbm.at[dst_all.at[pl.ds(t0, TAIL)]],
                     v0.at[pl.ds(0, TAIL)], sem0)
    pltpu.make_async_copy(u_hbm.at[src_all.at[pl.ds(t0, TAIL)]],
                          u0.at[pl.ds(0, TAIL)], sem0).wait()
    pltpu.make_async_copy(v_hbm.at[dst_all.at[pl.ds(t0, TAIL)]],
                          v0.at[pl.ds(0, TAIL)], sem0).wait()
    group16(t0, u0, v0, 0)

    pltpu.sync_copy(oe_v, out_e.at[pl.ds(base, ESUB)])
    pltpu.sync_copy(oa_v, out_a.at[pl.ds(base, ESUB)])
    pltpu.sync_copy(oq_v, out_q.at[pl.ds(base, ESUB)])


_edges = pl.kernel(
    _edge_body,
    out_type=[
        jax.ShapeDtypeStruct((E,), jnp.float32),
        jax.ShapeDtypeStruct((E,), jnp.float32),
        jax.ShapeDtypeStruct((E,), jnp.float32),
    ],
    mesh=plsc.VectorSubcoreMesh(core_axis_name="c", subcore_axis_name="s"),
    compiler_params=pltpu.CompilerParams(needs_layout_passes=False),
    scratch_types=[
        pltpu.VMEM((N,), jnp.float32),
        pltpu.VMEM((ESUB,), jnp.int32),
        pltpu.VMEM((ESUB,), jnp.int32),
        pltpu.VMEM((C, WP), jnp.int32),
        pltpu.VMEM((C, WP), jnp.int32),
        pltpu.VMEM((C, WP), jnp.int32),
        pltpu.VMEM((C, WP), jnp.int32),
        pltpu.VMEM((ESUB,), jnp.float32),
        pltpu.VMEM((ESUB,), jnp.float32),
        pltpu.VMEM((ESUB,), jnp.float32),
        pltpu.SemaphoreType.DMA,
        pltpu.SemaphoreType.DMA,
    ],
)


def _softmax_body(e_ref, a_ref, q_ref, o_ref):
    d_fk = jnp.sqrt(e_ref[...]) + BETA_POS * a_ref[...]
    s = jnp.exp(-d_fk) * q_ref[...] * (1.0 / math.sqrt(float(D)))
    m = jnp.max(s)
    ex = jnp.exp(s - m)
    o_ref[...] = ex / jnp.sum(ex)


_SR = 2500
_SC_COLS = 128


def _softmax(e2, a2, q2):
    return pl.pallas_call(
        _softmax_body,
        out_shape=jax.ShapeDtypeStruct((_SR, _SC_COLS), jnp.float32),
    )(e2, a2, q2)


def kernel(h, node_embeddings, edge_index, Wq, bq, Wk, bk, u_pos):
    src = edge_index[0]
    dst = edge_index[1]
    U, V, b2 = _prep(h, node_embeddings, Wq, bq.reshape(1, D),
                     Wk, bk.reshape(1, D), u_pos.reshape(D, 1))
    b = b2.reshape(N)
    euc2, asym, qk = _edges(U, V, b, src, dst)
    att = _softmax(euc2.reshape(_SR, _SC_COLS),
                   asym.reshape(_SR, _SC_COLS),
                   qk.reshape(_SR, _SC_COLS))
    return att.reshape(E)


# speedup vs baseline: 13.4080x; 1.0021x over previous
import functools
import math

import jax
import jax.numpy as jnp
from jax import lax
from jax.experimental import pallas as pl
from jax.experimental.pallas import tpu as pltpu
from jax.experimental.pallas import tpu_sc as plsc

N = 10000
E = 320000
D = 128
W = 2 * D
WP = D
BETA_POS = 0.1

NC = 2
NS = 16
NW = NC * NS
ESUB = E // NW
C = 64
NFULL = ESUB // C
TRIPLES = NFULL // 3
TAIL = ESUB - NFULL * C

_R = 1000


def _prep_body(h_ref, emb_ref, wq_ref, bq_ref, wk_ref, bk_ref, up_ref,
               u_ref, v_ref, b_ref):
    hh = h_ref[...]
    ee = emb_ref[...]
    dn = (((1,), (1,)), ((), ()))
    q = lax.dot_general(hh, wq_ref[...], dn,
                        preferred_element_type=jnp.float32,
                        precision=lax.Precision.HIGHEST) + bq_ref[...]
    k = lax.dot_general(hh, wk_ref[...], dn,
                        preferred_element_type=jnp.float32,
                        precision=lax.Precision.HIGHEST) + bk_ref[...]
    def pack_pair(xlo, xhi):
        lo = lax.bitcast_convert_type(xlo.astype(jnp.bfloat16),
                                      jnp.uint16).astype(jnp.uint32)
        hi = lax.bitcast_convert_type(xhi.astype(jnp.bfloat16),
                                      jnp.uint16).astype(jnp.uint32)
        return lax.bitcast_convert_type(lo | (hi << 16), jnp.int32)

    hd = D // 2
    u_ref[:, :hd] = pack_pair(ee[:, :hd], ee[:, hd:])
    u_ref[:, hd:] = pack_pair(q[:, :hd], q[:, hd:])
    v_ref[:, :hd] = pack_pair(ee[:, :hd], ee[:, hd:])
    v_ref[:, hd:] = pack_pair(k[:, :hd], k[:, hd:])
    b_ref[...] = lax.dot_general(ee, up_ref[...], (((1,), (0,)), ((), ())),
                                 preferred_element_type=jnp.float32,
                                 precision=lax.Precision.HIGHEST)


def _prep(h, emb, Wq, bq2, Wk, bk2, up2):
    grid = (N // _R,)
    row = lambda i: (i, 0)
    full = lambda i: (0, 0)
    return pl.pallas_call(
        _prep_body,
        grid=grid,
        in_specs=[
            pl.BlockSpec((_R, D), row),
            pl.BlockSpec((_R, D), row),
            pl.BlockSpec((D, D), full),
            pl.BlockSpec((1, D), full),
            pl.BlockSpec((D, D), full),
            pl.BlockSpec((1, D), full),
            pl.BlockSpec((D, 1), full),
        ],
        out_specs=[
            pl.BlockSpec((_R, WP), row),
            pl.BlockSpec((_R, WP), row),
            pl.BlockSpec((_R, 1), row),
        ],
        out_shape=[
            jax.ShapeDtypeStruct((N, WP), jnp.int32),
            jax.ShapeDtypeStruct((N, WP), jnp.int32),
            jax.ShapeDtypeStruct((N, 1), jnp.float32),
        ],
    )(h, emb, Wq, bq2, Wk, bk2, up2)


def _edge_body(u_hbm, v_hbm, b_hbm, src_hbm, dst_hbm,
               out_e, out_a, out_q,
               b_v, src_all, dst_all, u0, v0, u1, v1, u2, v2,
               oe_v, oa_v, oq_v, sem0, sem1, sem2):
    wid = lax.axis_index("s") * NC + lax.axis_index("c")
    base = wid * ESUB
    pltpu.sync_copy(src_hbm.at[pl.ds(base, 2 * C)], src_all.at[pl.ds(0, 2 * C)])
    pltpu.sync_copy(dst_hbm.at[pl.ds(base, 2 * C)], dst_all.at[pl.ds(0, 2 * C)])

    def fire(ci, ub, vb, sem):
        idx_s = src_all.at[pl.ds(ci * C, C)]
        idx_d = dst_all.at[pl.ds(ci * C, C)]
        pltpu.async_copy(u_hbm.at[idx_s], ub, sem)
        pltpu.async_copy(v_hbm.at[idx_d], vb, sem)

    def wait(ub, vb, sem):
        pltpu.make_async_copy(u_hbm.at[src_all.at[pl.ds(0, C)]], ub, sem).wait()
        pltpu.make_async_copy(v_hbm.at[dst_all.at[pl.ds(0, C)]], vb, sem).wait()

    lane = lax.broadcasted_iota(jnp.int32, (16,), 0)

    def expand(ref, r, k):
        w = ref[r, pl.ds(k * 16, 16)]
        hi = plsc.bitcast(w, jnp.float32)
        lo = plsc.bitcast(w << 16, jnp.float32)
        return hi, lo

    def group16(base_e, ub, vb, row0):
        res_e = jnp.zeros((16,), jnp.float32)
        res_q = jnp.zeros((16,), jnp.float32)
        for j in range(16):
            r = row0 + j
            acc_e = acc_q = None
            for k in range(4):
                uh, ul = expand(ub, r, k)
                vh, vl = expand(vb, r, k)
                dh = vh - uh
                dl = vl - ul
                t = dh * dh + dl * dl
                acc_e = t if acc_e is None else acc_e + t
            for k in range(4, 8):
                uh, ul = expand(ub, r, k)
                vh, vl = expand(vb, r, k)
                t = uh * vh + ul * vl
                acc_q = t if acc_q is None else acc_q + t
            mask = lane == j
            res_e = jnp.where(mask, jnp.sum(acc_e), res_e)
            res_q = jnp.where(mask, jnp.sum(acc_q), res_q)
        oe_v[pl.ds(base_e, 16)] = res_e
        oq_v[pl.ds(base_e, 16)] = res_q

    def asym16(e0):
        s16 = src_all[pl.ds(e0, 16)]
        d16 = dst_all[pl.ds(e0, 16)]
        bs = plsc.load_gather(b_v, [s16])
        bd = plsc.load_gather(b_v, [d16])
        oa_v[pl.ds(e0, 16)] = bd - bs

    def compute(ci, ub, vb):
        o0 = ci * C

        def gbody(g, carry):
            group16(o0 + g * 16, ub, vb, g * 16)
            asym16(o0 + g * 16)
            return carry

        lax.fori_loop(0, C // 16, gbody, 0)

    fire(0, u0, v0, sem0)
    fire(1, u1, v1, sem1)

    pltpu.sync_copy(src_hbm.at[pl.ds(base, ESUB)], src_all)
    pltpu.sync_copy(dst_hbm.at[pl.ds(base, ESUB)], dst_all)
    pltpu.sync_copy(b_hbm, b_v)

    def tri_body(t, carry):
        c0 = 3 * t
        wait(u0, v0, sem0)
        compute(c0, u0, v0)
        fire(c0 + 2, u2, v2, sem2)
        wait(u1, v1, sem1)
        compute(c0 + 1, u1, v1)

        @pl.when(t < TRIPLES - 1)
        def _():
            fire(c0 + 3, u0, v0, sem0)

        wait(u2, v2, sem2)
        compute(c0 + 2, u2, v2)

        @pl.when(t < TRIPLES - 1)
        def _():
            fire(c0 + 4, u1, v1, sem1)

        return carry

    lax.fori_loop(0, TRIPLES, tri_body, 0)

    t0 = NFULL * C
    pltpu.async_copy(u_hbm.at[src_all.at[pl.ds(t0, TAIL)]],
                     u0.at[pl.ds(0, TAIL)], sem0)
    pltpu.async_copy(v_hbm.at[dst_all.at[pl.ds(t0, TAIL)]],
                     v0.at[pl.ds(0, TAIL)], sem0)
    pltpu.make_async_copy(u_hbm.at[src_all.at[pl.ds(t0, TAIL)]],
                          u0.at[pl.ds(0, TAIL)], sem0).wait()
    pltpu.make_async_copy(v_hbm.at[dst_all.at[pl.ds(t0, TAIL)]],
                          v0.at[pl.ds(0, TAIL)], sem0).wait()
    group16(t0, u0, v0, 0)
    asym16(t0)

    pltpu.sync_copy(oe_v, out_e.at[pl.ds(base, ESUB)])
    pltpu.sync_copy(oa_v, out_a.at[pl.ds(base, ESUB)])
    pltpu.sync_copy(oq_v, out_q.at[pl.ds(base, ESUB)])


_edges = pl.kernel(
    _edge_body,
    out_type=[
        jax.ShapeDtypeStruct((E,), jnp.float32),
        jax.ShapeDtypeStruct((E,), jnp.float32),
        jax.ShapeDtypeStruct((E,), jnp.float32),
    ],
    mesh=plsc.VectorSubcoreMesh(core_axis_name="c", subcore_axis_name="s"),
    compiler_params=pltpu.CompilerParams(needs_layout_passes=False),
    scratch_types=[
        pltpu.VMEM((N,), jnp.float32),
        pltpu.VMEM((ESUB,), jnp.int32),
        pltpu.VMEM((ESUB,), jnp.int32),
        pltpu.VMEM((C, WP), jnp.int32),
        pltpu.VMEM((C, WP), jnp.int32),
        pltpu.VMEM((C, WP), jnp.int32),
        pltpu.VMEM((C, WP), jnp.int32),
        pltpu.VMEM((C, WP), jnp.int32),
        pltpu.VMEM((C, WP), jnp.int32),
        pltpu.VMEM((ESUB,), jnp.float32),
        pltpu.VMEM((ESUB,), jnp.float32),
        pltpu.VMEM((ESUB,), jnp.float32),
        pltpu.SemaphoreType.DMA,
        pltpu.SemaphoreType.DMA,
        pltpu.SemaphoreType.DMA,
    ],
)


def _softmax_body(e_ref, a_ref, q_ref, o_ref):
    d_fk = jnp.sqrt(e_ref[...]) + BETA_POS * a_ref[...]
    s = jnp.exp(-d_fk) * q_ref[...] * (1.0 / math.sqrt(float(D)))
    m = jnp.max(s)
    ex = jnp.exp(s - m)
    o_ref[...] = ex / jnp.sum(ex)


_SR = 2500
_SC_COLS = 128


def _softmax(e2, a2, q2):
    return pl.pallas_call(
        _softmax_body,
        out_shape=jax.ShapeDtypeStruct((_SR, _SC_COLS), jnp.float32),
    )(e2, a2, q2)


def kernel(h, node_embeddings, edge_index, Wq, bq, Wk, bk, u_pos):
    src = edge_index[0]
    dst = edge_index[1]
    U, V, b2 = _prep(h, node_embeddings, Wq, bq.reshape(1, D),
                     Wk, bk.reshape(1, D), u_pos.reshape(D, 1))
    b = b2.reshape(N)
    euc2, asym, qk = _edges(U, V, b, src, dst)
    att = _softmax(euc2.reshape(_SR, _SC_COLS),
                   asym.reshape(_SR, _SC_COLS),
                   qk.reshape(_SR, _SC_COLS))
    return att.reshape(E)


# speedup vs baseline: 13.9785x; 1.0425x over previous
import functools
import math

import jax
import jax.numpy as jnp
from jax import lax
from jax.experimental import pallas as pl
from jax.experimental.pallas import tpu as pltpu
from jax.experimental.pallas import tpu_sc as plsc

N = 10000
E = 320000
D = 128
W = 2 * D
WP = D
BETA_POS = 0.1

NC = 2
NS = 16
NW = NC * NS
ESUB = E // NW
C = 64
NFULL = ESUB // C
TRIPLES = NFULL // 3
INV_SCALE = 1.0 / math.sqrt(float(D))
TAIL = ESUB - NFULL * C

_R = 1000


def _prep_body(h_ref, emb_ref, wq_ref, bq_ref, wk_ref, bk_ref, up_ref,
               u_ref, v_ref, b_ref):
    hh = h_ref[...]
    ee = emb_ref[...]
    dn = (((1,), (1,)), ((), ()))
    q = lax.dot_general(hh, wq_ref[...], dn,
                        preferred_element_type=jnp.float32,
                        precision=lax.Precision.HIGHEST) + bq_ref[...]
    k = lax.dot_general(hh, wk_ref[...], dn,
                        preferred_element_type=jnp.float32,
                        precision=lax.Precision.HIGHEST) + bk_ref[...]
    def pack_pair(xlo, xhi):
        lo = lax.bitcast_convert_type(xlo.astype(jnp.bfloat16),
                                      jnp.uint16).astype(jnp.uint32)
        hi = lax.bitcast_convert_type(xhi.astype(jnp.bfloat16),
                                      jnp.uint16).astype(jnp.uint32)
        return lax.bitcast_convert_type(lo | (hi << 16), jnp.int32)

    hd = D // 2
    u_ref[:, :hd] = pack_pair(ee[:, :hd], ee[:, hd:])
    u_ref[:, hd:] = pack_pair(q[:, :hd], q[:, hd:])
    v_ref[:, :hd] = pack_pair(ee[:, :hd], ee[:, hd:])
    v_ref[:, hd:] = pack_pair(k[:, :hd], k[:, hd:])
    b_ref[...] = lax.dot_general(ee, up_ref[...], (((1,), (0,)), ((), ())),
                                 preferred_element_type=jnp.float32,
                                 precision=lax.Precision.HIGHEST)


def _prep(h, emb, Wq, bq2, Wk, bk2, up2):
    grid = (N // _R,)
    row = lambda i: (i, 0)
    full = lambda i: (0, 0)
    return pl.pallas_call(
        _prep_body,
        grid=grid,
        in_specs=[
            pl.BlockSpec((_R, D), row),
            pl.BlockSpec((_R, D), row),
            pl.BlockSpec((D, D), full),
            pl.BlockSpec((1, D), full),
            pl.BlockSpec((D, D), full),
            pl.BlockSpec((1, D), full),
            pl.BlockSpec((D, 1), full),
        ],
        out_specs=[
            pl.BlockSpec((_R, WP), row),
            pl.BlockSpec((_R, WP), row),
            pl.BlockSpec((_R, 1), row),
        ],
        out_shape=[
            jax.ShapeDtypeStruct((N, WP), jnp.int32),
            jax.ShapeDtypeStruct((N, WP), jnp.int32),
            jax.ShapeDtypeStruct((N, 1), jnp.float32),
        ],
    )(h, emb, Wq, bq2, Wk, bk2, up2)


def _edge_body(u_hbm, v_hbm, b_hbm, ei_hbm,
               out_s,
               b_v, src_all, dst_all, u0, v0, u1, v1, u2, v2,
               os_v, sem0, sem1, sem2):
    wid = lax.axis_index("s") * NC + lax.axis_index("c")
    base = wid * ESUB
    pltpu.sync_copy(ei_hbm.at[pl.ds(base, 2 * C)], src_all.at[pl.ds(0, 2 * C)])
    pltpu.sync_copy(ei_hbm.at[pl.ds(E + base, 2 * C)], dst_all.at[pl.ds(0, 2 * C)])

    def fire(ci, ub, vb, sem):
        idx_s = src_all.at[pl.ds(ci * C, C)]
        idx_d = dst_all.at[pl.ds(ci * C, C)]
        pltpu.async_copy(u_hbm.at[idx_s], ub, sem)
        pltpu.async_copy(v_hbm.at[idx_d], vb, sem)

    def wait(ub, vb, sem):
        pltpu.make_async_copy(u_hbm.at[src_all.at[pl.ds(0, C)]], ub, sem).wait()
        pltpu.make_async_copy(v_hbm.at[dst_all.at[pl.ds(0, C)]], vb, sem).wait()

    lane = lax.broadcasted_iota(jnp.int32, (16,), 0)

    def expand(ref, r, k):
        w = ref[r, pl.ds(k * 16, 16)]
        hi = plsc.bitcast(w, jnp.float32)
        lo = plsc.bitcast(w << 16, jnp.float32)
        return hi, lo

    def sqrt16(x):
        bits = plsc.bitcast(x, jnp.int32)
        y = plsc.bitcast(0x5F3759DF - (bits >> 1), jnp.float32)
        y = y * (1.5 - 0.5 * x * y * y)
        y = y * (1.5 - 0.5 * x * y * y)
        return x * y

    def group16(base_e, ub, vb, row0):
        res_e = jnp.zeros((16,), jnp.float32)
        res_q = jnp.zeros((16,), jnp.float32)
        for j in range(16):
            r = row0 + j
            acc_e = acc_q = None
            for k in range(4):
                uh, ul = expand(ub, r, k)
                vh, vl = expand(vb, r, k)
                dh = vh - uh
                dl = vl - ul
                t = dh * dh + dl * dl
                acc_e = t if acc_e is None else acc_e + t
            for k in range(4, 8):
                uh, ul = expand(ub, r, k)
                vh, vl = expand(vb, r, k)
                t = uh * vh + ul * vl
                acc_q = t if acc_q is None else acc_q + t
            mask = lane == j
            res_e = jnp.where(mask, jnp.sum(acc_e), res_e)
            res_q = jnp.where(mask, jnp.sum(acc_q), res_q)
        s16 = src_all[pl.ds(base_e, 16)]
        d16 = dst_all[pl.ds(base_e, 16)]
        bs = plsc.load_gather(b_v, [s16])
        bd = plsc.load_gather(b_v, [d16])
        d_fk = sqrt16(res_e) + BETA_POS * (bd - bs)
        os_v[pl.ds(base_e, 16)] = jnp.exp(-d_fk) * res_q * INV_SCALE

    def compute(ci, ub, vb):
        o0 = ci * C

        def gbody(g, carry):
            group16(o0 + g * 16, ub, vb, g * 16)
            return carry

        lax.fori_loop(0, C // 16, gbody, 0)

    fire(0, u0, v0, sem0)
    fire(1, u1, v1, sem1)

    pltpu.sync_copy(ei_hbm.at[pl.ds(base, ESUB)], src_all)
    pltpu.sync_copy(ei_hbm.at[pl.ds(E + base, ESUB)], dst_all)
    pltpu.sync_copy(b_hbm, b_v)

    def tri_body(t, carry):
        c0 = 3 * t
        wait(u0, v0, sem0)
        compute(c0, u0, v0)
        fire(c0 + 2, u2, v2, sem2)
        wait(u1, v1, sem1)
        compute(c0 + 1, u1, v1)

        @pl.when(t < TRIPLES - 1)
        def _():
            fire(c0 + 3, u0, v0, sem0)

        wait(u2, v2, sem2)
        compute(c0 + 2, u2, v2)

        @pl.when(t < TRIPLES - 1)
        def _():
            fire(c0 + 4, u1, v1, sem1)

        return carry

    lax.fori_loop(0, TRIPLES, tri_body, 0)

    t0 = NFULL * C
    pltpu.async_copy(u_hbm.at[src_all.at[pl.ds(t0, TAIL)]],
                     u0.at[pl.ds(0, TAIL)], sem0)
    pltpu.async_copy(v_hbm.at[dst_all.at[pl.ds(t0, TAIL)]],
                     v0.at[pl.ds(0, TAIL)], sem0)
    pltpu.make_async_copy(u_hbm.at[src_all.at[pl.ds(t0, TAIL)]],
                          u0.at[pl.ds(0, TAIL)], sem0).wait()
    pltpu.make_async_copy(v_hbm.at[dst_all.at[pl.ds(t0, TAIL)]],
                          v0.at[pl.ds(0, TAIL)], sem0).wait()
    group16(t0, u0, v0, 0)

    pltpu.sync_copy(os_v, out_s.at[pl.ds(base, ESUB)])


_edges = pl.kernel(
    _edge_body,
    out_type=jax.ShapeDtypeStruct((E,), jnp.float32),
    mesh=plsc.VectorSubcoreMesh(core_axis_name="c", subcore_axis_name="s"),
    compiler_params=pltpu.CompilerParams(needs_layout_passes=False),
    scratch_types=[
        pltpu.VMEM((N,), jnp.float32),
        pltpu.VMEM((ESUB,), jnp.int32),
        pltpu.VMEM((ESUB,), jnp.int32),
        pltpu.VMEM((C, WP), jnp.int32),
        pltpu.VMEM((C, WP), jnp.int32),
        pltpu.VMEM((C, WP), jnp.int32),
        pltpu.VMEM((C, WP), jnp.int32),
        pltpu.VMEM((C, WP), jnp.int32),
        pltpu.VMEM((C, WP), jnp.int32),
        pltpu.VMEM((ESUB,), jnp.float32),
        pltpu.SemaphoreType.DMA,
        pltpu.SemaphoreType.DMA,
        pltpu.SemaphoreType.DMA,
    ],
)


def _softmax_body(s_ref, o_ref):
    s = s_ref[...]
    m = jnp.max(s)
    ex = jnp.exp(s - m)
    o_ref[...] = ex / jnp.sum(ex)


_SR = 2500
_SC_COLS = 128


def _softmax(s2):
    return pl.pallas_call(
        _softmax_body,
        out_shape=jax.ShapeDtypeStruct((_SR, _SC_COLS), jnp.float32),
    )(s2)


def kernel(h, node_embeddings, edge_index, Wq, bq, Wk, bk, u_pos):
    U, V, b2 = _prep(h, node_embeddings, Wq, bq.reshape(1, D),
                     Wk, bk.reshape(1, D), u_pos.reshape(D, 1))
    s = _edges(U, V, b2.reshape(N), edge_index.reshape(2 * E))
    att = _softmax(s.reshape(_SR, _SC_COLS))
    return att.reshape(E)
